# baseline (speedup 1.0000x reference)
import functools

import jax
import jax.numpy as jnp
from jax import lax
from jax.experimental import pallas as pl
from jax.experimental.pallas import tpu as pltpu
from jax.experimental.pallas import tpu_sc as plsc

N_BITS = 20
LO_BITS = 10
C_ROWS = 2048
D = 64
BLK = 256
L = 16


def _combine_tables_body(table_ref, out_ref):
    K = N_BITS + 1
    k = lax.broadcasted_iota(jnp.int32, (K, C_ROWS), 0)
    r = lax.broadcasted_iota(jnp.int32, (K, C_ROWS), 1)
    g = r >> LO_BITS
    local = r & ((1 << LO_BITS) - 1)
    shift = k - g * LO_BITS
    in_group = (shift >= 0) & (shift < LO_BITS)
    bits_t = jnp.where(in_group, (local >> jnp.maximum(shift, 0)) & 1, 0)
    out_ref[...] = lax.dot_general(
        table_ref[...], bits_t.astype(jnp.float32),
        dimension_numbers=(((0,), (0,)), ((), ())),
        preferred_element_type=jnp.float32)


def _combine_tables(table):
    return pl.pallas_call(
        _combine_tables_body,
        out_shape=jax.ShapeDtypeStruct((D, C_ROWS), jnp.float32),
    )(table)


def _pack_pairs(ct):
    ct_bf = ct.astype(jnp.bfloat16)
    pairs = ct_bf.reshape(D // 2, 2, C_ROWS).transpose(0, 2, 1)
    return lax.bitcast_convert_type(pairs, jnp.int32)


def _sc_lookup(ctab_packed, xf):
    info = plsc.get_sparse_core_info()
    nc, ns = info.num_cores, info.num_subcores
    nw = nc * ns
    t_total = xf.shape[0]
    per_w = t_total // nw
    nblk = per_w // BLK
    assert per_w * nw == t_total and nblk * BLK == per_w and nblk % 2 == 0

    mesh = plsc.VectorSubcoreMesh(core_axis_name="c", subcore_axis_name="s")

    @functools.partial(
        pl.kernel,
        mesh=mesh,
        compiler_params=pltpu.CompilerParams(needs_layout_passes=False),
        out_type=jax.ShapeDtypeStruct((D, t_total), jnp.float32),
        scratch_types=[
            pltpu.VMEM((D // 2, C_ROWS), jnp.int32),
            pltpu.VMEM((2, BLK), jnp.int32),
            pltpu.VMEM((2, D, BLK), jnp.float32),
            pltpu.SemaphoreType.DMA,
            pltpu.SemaphoreType.DMA,
            pltpu.SemaphoreType.DMA,
            pltpu.SemaphoreType.DMA,
        ],
    )
    def lookup(ctab_hbm, x_hbm, out_hbm, tab, xv, ov, xs0, xs1, os0, os1):
        xsem = (xs0, xs1)
        osem = (os0, os1)
        wid = lax.axis_index("s") * nc + lax.axis_index("c")
        base = wid * per_w

        pltpu.sync_copy(ctab_hbm, tab)

        def fire_x(j, s):
            pltpu.async_copy(x_hbm.at[pl.ds(base + j * BLK, BLK)],
                             xv.at[s], xsem[s])

        fire_x(0, 0)

        def blockstep(j, s):
            pltpu.make_async_copy(x_hbm.at[pl.ds(0, BLK)], xv.at[s],
                                  xsem[s]).wait()

            @pl.when(j + 1 < nblk)
            def _():
                fire_x(j + 1, 1 - s)

            @pl.when(j >= 2)
            def _():
                pltpu.make_async_copy(ov.at[s],
                                      out_hbm.at[:, pl.ds(0, BLK)],
                                      osem[s]).wait()

            @plsc.parallel_loop(0, BLK // L, unroll=8)
            def group(g):
                xs16 = xv[s, pl.ds(g * L, L)]
                i_lo = xs16 & (C_ROWS // 2 - 1)
                i_hi = (xs16 >> LO_BITS) + C_ROWS // 2
                for p in range(D // 2):
                    cp = jnp.full((L,), p, jnp.int32)
                    wlo = plsc.load_gather(tab, [cp, i_lo])
                    whi = plsc.load_gather(tab, [cp, i_hi])
                    bsum = (plsc.bitcast(wlo, jnp.bfloat16)
                            + plsc.bitcast(whi, jnp.bfloat16))
                    e, o = plsc.unpack(
                        bsum, format=plsc.PackFormat.INTERLEAVED,
                        preferred_element_type=jnp.float32)
                    ov[s, 2 * p, pl.ds(g * L, L)] = e
                    ov[s, 2 * p + 1, pl.ds(g * L, L)] = o

            pltpu.async_copy(
                ov.at[s], out_hbm.at[:, pl.ds(base + j * BLK, BLK)], osem[s])

        def pair(jj, carry):
            blockstep(2 * jj, 0)
            blockstep(2 * jj + 1, 1)
            return carry

        lax.fori_loop(0, nblk // 2, pair, 0)

        for s in range(2):
            pltpu.make_async_copy(ov.at[s], out_hbm.at[:, pl.ds(0, BLK)],
                                  osem[s]).wait()

    return lookup(ctab_packed, xf)


def kernel(x, table):
    xf = x.reshape(-1)
    ctab_packed = _pack_pairs(_combine_tables(table))
    out_t = _sc_lookup(ctab_packed, xf)
    return out_t.T


# speedup vs baseline: 1.3547x; 1.3547x over previous
import functools

import jax
import jax.numpy as jnp
from jax import lax
from jax.experimental import pallas as pl
from jax.experimental.pallas import tpu as pltpu
from jax.experimental.pallas import tpu_sc as plsc

N_BITS = 20
LO_BITS = 10
C_ROWS = 2048
D = 64
BLK = 256
L = 16


def _combine_tables_body(table_ref, out_ref):
    K = N_BITS + 1
    k = lax.broadcasted_iota(jnp.int32, (K, C_ROWS), 0)
    r = lax.broadcasted_iota(jnp.int32, (K, C_ROWS), 1)
    g = r >> LO_BITS
    local = r & ((1 << LO_BITS) - 1)
    shift = k - g * LO_BITS
    in_group = (shift >= 0) & (shift < LO_BITS)
    bits_t = jnp.where(in_group, (local >> jnp.maximum(shift, 0)) & 1, 0)
    out_ref[...] = lax.dot_general(
        table_ref[...], bits_t.astype(jnp.float32),
        dimension_numbers=(((0,), (0,)), ((), ())),
        preferred_element_type=jnp.float32)


def _combine_tables(table):
    return pl.pallas_call(
        _combine_tables_body,
        out_shape=jax.ShapeDtypeStruct((D, C_ROWS), jnp.float32),
    )(table)


def _pack_pairs(ct):
    ct_bf = ct.astype(jnp.bfloat16)
    pairs = ct_bf.reshape(D // 2, 2, C_ROWS).transpose(0, 2, 1)
    packed = lax.bitcast_convert_type(pairs, jnp.int32)
    return packed.reshape(-1)


def _sc_lookup(ctab_packed, xf):
    info = plsc.get_sparse_core_info()
    nc, ns = info.num_cores, info.num_subcores
    nw = nc * ns
    t_total = xf.shape[0]
    per_w = t_total // nw
    nblk = per_w // BLK
    assert per_w * nw == t_total and nblk * BLK == per_w and nblk % 2 == 0

    mesh = plsc.VectorSubcoreMesh(core_axis_name="c", subcore_axis_name="s")

    @functools.partial(
        pl.kernel,
        mesh=mesh,
        compiler_params=pltpu.CompilerParams(needs_layout_passes=False),
        out_type=jax.ShapeDtypeStruct((D, t_total), jnp.float32),
        scratch_types=[
            pltpu.VMEM((D // 2 * C_ROWS,), jnp.int32),
            pltpu.VMEM((2, BLK), jnp.int32),
            pltpu.VMEM((2, D, BLK), jnp.float32),
            pltpu.SemaphoreType.DMA,
            pltpu.SemaphoreType.DMA,
            pltpu.SemaphoreType.DMA,
            pltpu.SemaphoreType.DMA,
        ],
    )
    def lookup(ctab_hbm, x_hbm, out_hbm, tab, xv, ov, xs0, xs1, os0, os1):
        xsem = (xs0, xs1)
        osem = (os0, os1)
        wid = lax.axis_index("s") * nc + lax.axis_index("c")
        base = wid * per_w

        pltpu.sync_copy(ctab_hbm, tab)

        def fire_x(j, s):
            pltpu.async_copy(x_hbm.at[pl.ds(base + j * BLK, BLK)],
                             xv.at[s], xsem[s])

        fire_x(0, 0)

        def blockstep(j, s):
            pltpu.make_async_copy(x_hbm.at[pl.ds(0, BLK)], xv.at[s],
                                  xsem[s]).wait()

            @pl.when(j + 1 < nblk)
            def _():
                fire_x(j + 1, 1 - s)

            @pl.when(j >= 2)
            def _():
                pltpu.make_async_copy(ov.at[s],
                                      out_hbm.at[:, pl.ds(0, BLK)],
                                      osem[s]).wait()

            @plsc.parallel_loop(0, BLK // L, unroll=4)
            def group(g):
                xs16 = xv[s, pl.ds(g * L, L)]
                i_lo = xs16 & (C_ROWS // 2 - 1)
                i_hi = (xs16 >> LO_BITS) + C_ROWS // 2
                for p in range(D // 2):
                    wlo = plsc.load_gather(tab, [i_lo + p * C_ROWS])
                    whi = plsc.load_gather(tab, [i_hi + p * C_ROWS])
                    bsum = (plsc.bitcast(wlo, jnp.bfloat16)
                            + plsc.bitcast(whi, jnp.bfloat16))
                    e, o = plsc.unpack(
                        bsum, format=plsc.PackFormat.INTERLEAVED,
                        preferred_element_type=jnp.float32)
                    ov[s, 2 * p, pl.ds(g * L, L)] = e
                    ov[s, 2 * p + 1, pl.ds(g * L, L)] = o

            pltpu.async_copy(
                ov.at[s], out_hbm.at[:, pl.ds(base + j * BLK, BLK)], osem[s])

        def pair(jj, carry):
            blockstep(2 * jj, 0)
            blockstep(2 * jj + 1, 1)
            return carry

        lax.fori_loop(0, nblk // 2, pair, 0)

        for s in range(2):
            pltpu.make_async_copy(ov.at[s], out_hbm.at[:, pl.ds(0, BLK)],
                                  osem[s]).wait()

    return lookup(ctab_packed, xf)


def kernel(x, table):
    xf = x.reshape(-1)
    ctab_packed = _pack_pairs(_combine_tables(table))
    out_t = _sc_lookup(ctab_packed, xf)
    return out_t.T
